# baseline (speedup 1.0000x reference)
import jax
import jax.numpy as jnp
import numpy as np
from jax.experimental import pallas as pl
from jax.experimental.pallas import tpu as pltpu

_MASK_PROB = 0.065
_MASK_LENGTH = 10
_TBLK = 1024

_starts_cache = {}


def _span_starts(B, T):
    if (B, T) not in _starts_cache:
        n = int(_MASK_PROB * T / _MASK_LENGTH)
        with jax.ensure_compile_time_eval():
            key = jax.random.key(42)
            rows = []
            for b in range(B):
                kb = jax.random.fold_in(key, b)
                rows.append(np.asarray(jax.random.randint(kb, (n,), 0, T - _MASK_LENGTH)))
        _starts_cache[(B, T)] = np.stack(rows).astype(np.int32)
    return _starts_cache[(B, T)]


def _masked_fill_kernel(starts_ref, x_ref, emb_ref, out_ref, mask_ref, *, tblk, nspans):
    b = pl.program_id(0)
    t0 = pl.program_id(1) * tblk
    trow = jax.lax.broadcasted_iota(jnp.int32, (tblk, 1), 0) + t0
    mrow = jnp.zeros((tblk, 1), jnp.bool_)
    for s in range(nspans):
        st = starts_ref[b, s]
        mrow = mrow | ((trow >= st) & (trow < st + _MASK_LENGTH))
    out_ref[0] = jnp.where(mrow, emb_ref[...], x_ref[0])
    mask_ref[0] = mrow.astype(jnp.int32)


def kernel(x, mask_embedding):
    B, T, C = x.shape
    starts = _span_starts(B, T)
    nspans = starts.shape[1]
    tblk = _TBLK

    grid_spec = pltpu.PrefetchScalarGridSpec(
        num_scalar_prefetch=1,
        grid=(B, T // tblk),
        in_specs=[
            pl.BlockSpec((1, tblk, C), lambda b, t, s: (b, t, 0)),
            pl.BlockSpec((1, C), lambda b, t, s: (0, 0)),
        ],
        out_specs=[
            pl.BlockSpec((1, tblk, C), lambda b, t, s: (b, t, 0)),
            pl.BlockSpec((1, tblk, 1), lambda b, t, s: (b, t, 0)),
        ],
    )
    import functools
    body = functools.partial(_masked_fill_kernel, tblk=tblk, nspans=nspans)
    x_masked, mask_i32 = pl.pallas_call(
        body,
        grid_spec=grid_spec,
        out_shape=[
            jax.ShapeDtypeStruct((B, T, C), x.dtype),
            jax.ShapeDtypeStruct((B, T, 1), jnp.int32),
        ],
    )(starts, x, mask_embedding.reshape(1, C))
    return (x_masked, mask_i32.reshape(B, T).astype(bool))


# speedup vs baseline: 1.0427x; 1.0427x over previous
import jax
import jax.numpy as jnp
import numpy as np
from jax.experimental import pallas as pl
from jax.experimental.pallas import tpu as pltpu

_MASK_PROB = 0.065
_MASK_LENGTH = 10
_TBLK = 1024

_M32 = np.uint64(0xFFFFFFFF)


def _threefry2x32(k0, k1, x0, x1):
    def rotl(x, r):
        return ((x << np.uint64(r)) | (x >> np.uint64(32 - r))) & _M32

    ks = [np.uint64(k0), np.uint64(k1),
          np.uint64(k0) ^ np.uint64(k1) ^ np.uint64(0x1BD11BDA)]
    rotations = [(13, 15, 26, 6), (17, 29, 16, 24)]
    x0 = (x0 + ks[0]) & _M32
    x1 = (x1 + ks[1]) & _M32
    for i in range(5):
        for r in rotations[i % 2]:
            x0 = (x0 + x1) & _M32
            x1 = rotl(x1, r)
            x1 = x1 ^ x0
        x0 = (x0 + ks[(i + 1) % 3]) & _M32
        x1 = (x1 + ks[(i + 2) % 3] + np.uint64(i + 1)) & _M32
    return x0, x1


def _np_fold_in(k, data):
    o0, o1 = _threefry2x32(k[0], k[1],
                           np.array([data >> 32], np.uint64),
                           np.array([data & 0xFFFFFFFF], np.uint64))
    return (int(o0[0]), int(o1[0]))


def _np_random_bits(k, n):
    b0, b1 = _threefry2x32(k[0], k[1],
                           np.zeros(n, np.uint64),
                           np.arange(n, dtype=np.uint64))
    return (b0 ^ b1).astype(np.uint64)


def _np_randint(k, n, minval, maxval):
    b0, b1 = _threefry2x32(k[0], k[1],
                           np.zeros(2, np.uint64),
                           np.arange(2, dtype=np.uint64))
    k1, k2 = (int(b0[0]), int(b1[0])), (int(b0[1]), int(b1[1]))
    higher, lower = _np_random_bits(k1, n), _np_random_bits(k2, n)
    span = np.uint64(maxval - minval)
    mult = (np.uint64(2**16) % span)
    mult = (mult * mult) % span & _M32
    off = (((higher % span) * mult + (lower % span)) & _M32) % span
    return (np.int64(minval) + off.astype(np.int64)).astype(np.int32)


_starts_cache = {}


def _span_starts(B, T):
    if (B, T) not in _starts_cache:
        n = int(_MASK_PROB * T / _MASK_LENGTH)
        rows = [_np_randint(_np_fold_in((0, 42), b), n, 0, T - _MASK_LENGTH)
                for b in range(B)]
        _starts_cache[(B, T)] = np.stack(rows).astype(np.int32)
    return _starts_cache[(B, T)]


def _masked_fill_kernel(starts_ref, x_ref, emb_ref, out_ref, mask_ref, *, tblk, nspans):
    b = pl.program_id(0)
    t0 = pl.program_id(1) * tblk
    trow = jax.lax.broadcasted_iota(jnp.int32, (tblk, 1), 0) + t0
    tlane = jax.lax.broadcasted_iota(jnp.int32, (1, tblk), 1) + t0
    mrow = jnp.zeros((tblk, 1), jnp.bool_)
    mlane = jnp.zeros((1, tblk), jnp.bool_)
    for s in range(nspans):
        st = starts_ref[b, s]
        mrow = mrow | ((trow >= st) & (trow < st + _MASK_LENGTH))
        mlane = mlane | ((tlane >= st) & (tlane < st + _MASK_LENGTH))
    out_ref[0] = jnp.where(mrow, emb_ref[...], x_ref[0])
    mask_ref[0] = mlane.astype(jnp.int32)


def kernel(x, mask_embedding):
    B, T, C = x.shape
    starts = _span_starts(B, T)
    nspans = starts.shape[1]
    tblk = _TBLK

    grid_spec = pltpu.PrefetchScalarGridSpec(
        num_scalar_prefetch=1,
        grid=(B, T // tblk),
        in_specs=[
            pl.BlockSpec((1, tblk, C), lambda b, t, s: (b, t, 0)),
            pl.BlockSpec((1, C), lambda b, t, s: (0, 0)),
        ],
        out_specs=[
            pl.BlockSpec((1, tblk, C), lambda b, t, s: (b, t, 0)),
            pl.BlockSpec((1, 1, tblk), lambda b, t, s: (b, 0, t)),
        ],
    )
    import functools
    body = functools.partial(_masked_fill_kernel, tblk=tblk, nspans=nspans)
    x_masked, mask_i32 = pl.pallas_call(
        body,
        grid_spec=grid_spec,
        out_shape=[
            jax.ShapeDtypeStruct((B, T, C), x.dtype),
            jax.ShapeDtypeStruct((B, 1, T), jnp.int32),
        ],
    )(starts, x, mask_embedding.reshape(1, C))
    return (x_masked, mask_i32.reshape(B, T).astype(bool))


# speedup vs baseline: 1.1529x; 1.1057x over previous
import jax
import jax.numpy as jnp
import numpy as np
from jax.experimental import pallas as pl
from jax.experimental.pallas import tpu as pltpu

_MASK_PROB = 0.065
_MASK_LENGTH = 10
_TBLK = 1024

_M32 = np.uint64(0xFFFFFFFF)


def _threefry2x32(k0, k1, x0, x1):
    def rotl(x, r):
        return ((x << np.uint64(r)) | (x >> np.uint64(32 - r))) & _M32

    ks = [np.uint64(k0), np.uint64(k1),
          np.uint64(k0) ^ np.uint64(k1) ^ np.uint64(0x1BD11BDA)]
    rotations = [(13, 15, 26, 6), (17, 29, 16, 24)]
    x0 = (x0 + ks[0]) & _M32
    x1 = (x1 + ks[1]) & _M32
    for i in range(5):
        for r in rotations[i % 2]:
            x0 = (x0 + x1) & _M32
            x1 = rotl(x1, r)
            x1 = x1 ^ x0
        x0 = (x0 + ks[(i + 1) % 3]) & _M32
        x1 = (x1 + ks[(i + 2) % 3] + np.uint64(i + 1)) & _M32
    return x0, x1


def _np_fold_in(k, data):
    o0, o1 = _threefry2x32(k[0], k[1],
                           np.array([data >> 32], np.uint64),
                           np.array([data & 0xFFFFFFFF], np.uint64))
    return (int(o0[0]), int(o1[0]))


def _np_random_bits(k, n):
    b0, b1 = _threefry2x32(k[0], k[1],
                           np.zeros(n, np.uint64),
                           np.arange(n, dtype=np.uint64))
    return (b0 ^ b1).astype(np.uint64)


def _np_randint(k, n, minval, maxval):
    b0, b1 = _threefry2x32(k[0], k[1],
                           np.zeros(2, np.uint64),
                           np.arange(2, dtype=np.uint64))
    k1, k2 = (int(b0[0]), int(b1[0])), (int(b0[1]), int(b1[1]))
    higher, lower = _np_random_bits(k1, n), _np_random_bits(k2, n)
    span = np.uint64(maxval - minval)
    mult = (np.uint64(2**16) % span)
    mult = (mult * mult) % span & _M32
    off = (((higher % span) * mult + (lower % span)) & _M32) % span
    return (np.int64(minval) + off.astype(np.int64)).astype(np.int32)


_starts_cache = {}


def _span_starts(B, T):
    if (B, T) not in _starts_cache:
        n = int(_MASK_PROB * T / _MASK_LENGTH)
        rows = [_np_randint(_np_fold_in((0, 42), b), n, 0, T - _MASK_LENGTH)
                for b in range(B)]
        _starts_cache[(B, T)] = np.stack(rows).astype(np.int32)
    return _starts_cache[(B, T)]


def _masked_fill_kernel(starts_ref, x_ref, emb_ref, out_ref, mask_ref, *, tblk, nspans):
    b = pl.program_id(0)
    t0 = pl.program_id(1) * tblk
    tlane = jax.lax.broadcasted_iota(jnp.int32, (1, tblk), 1) + t0
    mlane = jnp.zeros((1, tblk), jnp.bool_)
    for s in range(nspans):
        st = starts_ref[b, s]
        mlane = mlane | ((tlane >= st) & (tlane < st + _MASK_LENGTH))
    mlane_i32 = mlane.astype(jnp.int32)
    mask_ref[0] = mlane_i32
    mrow = mlane_i32.reshape(tblk, 1) != 0
    out_ref[0] = jnp.where(mrow, emb_ref[...], x_ref[0])


def kernel(x, mask_embedding):
    B, T, C = x.shape
    starts = _span_starts(B, T)
    nspans = starts.shape[1]
    tblk = _TBLK

    grid_spec = pltpu.PrefetchScalarGridSpec(
        num_scalar_prefetch=1,
        grid=(B, T // tblk),
        in_specs=[
            pl.BlockSpec((1, tblk, C), lambda b, t, s: (b, t, 0)),
            pl.BlockSpec((1, C), lambda b, t, s: (0, 0)),
        ],
        out_specs=[
            pl.BlockSpec((1, tblk, C), lambda b, t, s: (b, t, 0)),
            pl.BlockSpec((1, 1, tblk), lambda b, t, s: (b, 0, t)),
        ],
    )
    import functools
    body = functools.partial(_masked_fill_kernel, tblk=tblk, nspans=nspans)
    x_masked, mask_i32 = pl.pallas_call(
        body,
        grid_spec=grid_spec,
        out_shape=[
            jax.ShapeDtypeStruct((B, T, C), x.dtype),
            jax.ShapeDtypeStruct((B, 1, T), jnp.int32),
        ],
    )(starts, x, mask_embedding.reshape(1, C))
    return (x_masked, mask_i32.reshape(B, T).astype(bool))
